# baseline (speedup 1.0000x reference)
import functools

import jax
import jax.numpy as jnp
from jax import lax
from jax.experimental import pallas as pl
from jax.experimental.pallas import tpu as pltpu
from jax.experimental.pallas import tpu_sc as plsc

_THRESH_NLOG = 0.35667494393873245
_N_MIN = 65536

_N, _C, _H, _W = 4, 19, 512, 512
_NPIX = _N * _H * _W

_ROWS = 64
_RGRID = _H // _ROWS


def _loss_body(logits_ref, labels_ref, loss_ref, cnt_ref, sum_ref):
    x = logits_ref[0]
    m = jnp.max(x, axis=0)
    e = jnp.exp(x - m[None])
    s = jnp.sum(e, axis=0)
    lse = m + jnp.log(s)
    lbl = labels_ref[0]
    cls = lax.broadcasted_iota(jnp.int32, x.shape, 0)
    pick = jnp.sum(jnp.where(cls == lbl[None], x, 0.0), axis=0)
    loss = jnp.maximum(lse - pick, 0.0)
    loss_ref[0] = loss
    mask = loss > _THRESH_NLOG
    c = jnp.sum(mask.astype(jnp.float32))
    sv = jnp.sum(jnp.where(mask, loss, 0.0))
    first = jnp.logical_and(pl.program_id(0) == 0, pl.program_id(1) == 0)

    @pl.when(first)
    def _():
        cnt_ref[0, 0] = c
        sum_ref[0, 0] = sv

    @pl.when(jnp.logical_not(first))
    def _():
        cnt_ref[0, 0] = cnt_ref[0, 0] + c
        sum_ref[0, 0] = sum_ref[0, 0] + sv


_loss_call = pl.pallas_call(
    _loss_body,
    grid=(_N, _RGRID),
    in_specs=[
        pl.BlockSpec((1, _C, _ROWS, _W), lambda n, r: (n, 0, r, 0)),
        pl.BlockSpec((1, _ROWS, _W), lambda n, r: (n, r, 0)),
    ],
    out_specs=[
        pl.BlockSpec((1, _ROWS, _W), lambda n, r: (n, r, 0)),
        pl.BlockSpec((1, 1), lambda n, r: (0, 0), memory_space=pltpu.SMEM),
        pl.BlockSpec((1, 1), lambda n, r: (0, 0), memory_space=pltpu.SMEM),
    ],
    out_shape=[
        jax.ShapeDtypeStruct((_N, _H, _W), jnp.float32),
        jax.ShapeDtypeStruct((1, 1), jnp.float32),
        jax.ShapeDtypeStruct((1, 1), jnp.float32),
    ],
)

_NC, _NS = 2, 16
_NW = _NC * _NS
_PER_W = _NPIX // _NW
_CH = 4096
_NCHUNK = _PER_W // _CH
_NPAIR = _NCHUNK // 2
_NBINS = 65536
_NREP = 8
_BINS_PER_TILE = _NREP * _NBINS // _NS


def _sc_hist_body(loss_hbm, param_hbm, zeros_hbm, cnt_out, psum_out,
                  pbuf, dbuf_a, dbuf_b, ibuf_a, ibuf_b, cbuf_a, cbuf_b,
                  abuf, cnt_sh, sem_a, sem_b):
    c = lax.axis_index("c")
    s = lax.axis_index("s")
    wid = s * _NC + c
    sl = pl.ds(s * _BINS_PER_TILE, _BINS_PER_TILE)
    pltpu.sync_copy(zeros_hbm.at[sl], cnt_sh.at[sl])
    pltpu.sync_copy(param_hbm, pbuf)
    plsc.subcore_barrier()

    p = pbuf[...]
    is_p1 = p < 0
    base = wid * _PER_W
    lanes = lax.iota(jnp.int32, 16)
    rep_off = (s % _NREP) * _NBINS
    abuf[...] = jnp.zeros((16,), jnp.float32)
    dbufs = (dbuf_a, dbuf_b)
    ibufs = (ibuf_a, ibuf_b)
    cbufs = (cbuf_a, cbuf_b)
    sems = (sem_a, sem_b)

    def process(off0, db, ib, cb):
        def vec_body(k, acc):
            for u in range(4):
                off = k * 64 + u * 16
                v = db[pl.ds(off, 16)]
                bits = lax.bitcast_convert_type(v, jnp.int32)
                hi = lax.shift_right_logical(bits, 16)
                lo = lax.bitwise_and(bits, jnp.int32(0xFFFF))
                match = jnp.logical_or(is_p1, hi == p)
                pos = lax.bitwise_and(off0 + off + lanes, jnp.int32(0xFFFF))
                idx = jnp.where(match, jnp.where(is_p1, hi, lo), pos) + rep_off
                ib[pl.ds(off, 16)] = idx
                cb[pl.ds(off, 16)] = jnp.where(match, 1.0, 0.0)
                acc = acc + jnp.where(hi > p, v, 0.0)
            return acc

        acc = lax.fori_loop(0, _CH // 64, vec_body, jnp.zeros((16,), jnp.float32))
        abuf[...] = abuf[...] + acc
        pltpu.sync_copy(cb, cnt_sh.at[ib], add=True)

    def start_load(ci, which):
        ci = jnp.minimum(ci, _NCHUNK - 1)
        pltpu.async_copy(loss_hbm.at[pl.ds(base + ci * _CH, _CH)],
                         dbufs[which], sems[which])

    def wait_load(which):
        pltpu.make_async_copy(loss_hbm.at[pl.ds(0, _CH)],
                              dbufs[which], sems[which]).wait()

    start_load(0, 0)

    def pair_body(i, carry):
        start_load(2 * i + 1, 1)
        wait_load(0)
        process(base + (2 * i) * _CH, dbuf_a, ibuf_a, cbuf_a)
        start_load(2 * i + 2, 0)
        wait_load(1)
        process(base + (2 * i + 1) * _CH, dbuf_b, ibuf_b, cbuf_b)
        return carry

    lax.fori_loop(0, _NPAIR, pair_body, 0)
    wait_load(0)
    pltpu.sync_copy(abuf, psum_out.at[c, s])
    plsc.subcore_barrier()

    @pl.when(s == 0)
    def _():
        pltpu.sync_copy(cnt_sh, cnt_out.at[c])


def _sc_sum_body(loss_hbm, param_hbm, psum_out,
                 pbuf, dbuf_a, dbuf_b, abuf, sem_a, sem_b):
    c = lax.axis_index("c")
    s = lax.axis_index("s")
    wid = s * _NC + c
    pltpu.sync_copy(param_hbm, pbuf)
    b1 = pbuf[pl.ds(0, 16)]
    b2 = pbuf[pl.ds(16, 16)]
    base = wid * _PER_W
    dbufs = (dbuf_a, dbuf_b)
    sems = (sem_a, sem_b)

    def process(db, acc0):
        def vec_body(k, acc):
            for u in range(4):
                off = k * 64 + u * 16
                v = db[pl.ds(off, 16)]
                bits = lax.bitcast_convert_type(v, jnp.int32)
                hi = lax.shift_right_logical(bits, 16)
                lo = lax.bitwise_and(bits, jnp.int32(0xFFFF))
                m = jnp.logical_and(hi == b1, lo > b2)
                acc = acc + jnp.where(m, v, 0.0)
            return acc

        return lax.fori_loop(0, _CH // 64, vec_body, acc0)

    def start_load(ci, which):
        ci = jnp.minimum(ci, _NCHUNK - 1)
        pltpu.async_copy(loss_hbm.at[pl.ds(base + ci * _CH, _CH)],
                         dbufs[which], sems[which])

    def wait_load(which):
        pltpu.make_async_copy(loss_hbm.at[pl.ds(0, _CH)],
                              dbufs[which], sems[which]).wait()

    start_load(0, 0)

    def pair_body(i, acc):
        start_load(2 * i + 1, 1)
        wait_load(0)
        acc = process(dbuf_a, acc)
        start_load(2 * i + 2, 0)
        wait_load(1)
        acc = process(dbuf_b, acc)
        return acc

    acc = lax.fori_loop(0, _NPAIR, pair_body, jnp.zeros((16,), jnp.float32))
    wait_load(0)
    abuf[...] = acc
    pltpu.sync_copy(abuf, psum_out.at[c, s])


@functools.lru_cache(maxsize=1)
def _get_hist_call():
    return pl.kernel(
        _sc_hist_body,
        mesh=plsc.VectorSubcoreMesh(core_axis_name="c", subcore_axis_name="s",
                                    num_cores=_NC, num_subcores=_NS),
        out_type=[
            jax.ShapeDtypeStruct((_NC, _NREP * _NBINS), jnp.float32),
            jax.ShapeDtypeStruct((_NC, _NS, 16), jnp.float32),
        ],
        scratch_types=[
            pltpu.VMEM((16,), jnp.int32),
            pltpu.VMEM((_CH,), jnp.float32),
            pltpu.VMEM((_CH,), jnp.float32),
            pltpu.VMEM((_CH,), jnp.int32),
            pltpu.VMEM((_CH,), jnp.int32),
            pltpu.VMEM((_CH,), jnp.float32),
            pltpu.VMEM((_CH,), jnp.float32),
            pltpu.VMEM((16,), jnp.float32),
            pltpu.VMEM_SHARED((_NREP * _NBINS,), jnp.float32),
            pltpu.SemaphoreType.DMA,
            pltpu.SemaphoreType.DMA,
        ],
    )


@functools.lru_cache(maxsize=1)
def _get_sum_call():
    return pl.kernel(
        _sc_sum_body,
        mesh=plsc.VectorSubcoreMesh(core_axis_name="c", subcore_axis_name="s",
                                    num_cores=_NC, num_subcores=_NS),
        out_type=jax.ShapeDtypeStruct((_NC, _NS, 16), jnp.float32),
        scratch_types=[
            pltpu.VMEM((32,), jnp.int32),
            pltpu.VMEM((_CH,), jnp.float32),
            pltpu.VMEM((_CH,), jnp.float32),
            pltpu.VMEM((16,), jnp.float32),
            pltpu.SemaphoreType.DMA,
            pltpu.SemaphoreType.DMA,
        ],
    )

_HR, _HCOL = 512, 128


def _scan_body(cnt_ref, k_ref, b_ref, above_ref):
    C = jnp.sum(cnt_ref[...], axis=0)
    rr = lax.broadcasted_iota(jnp.int32, (_HCOL, _HCOL), 0)
    cc = lax.broadcasted_iota(jnp.int32, (_HCOL, _HCOL), 1)
    t_strict = (rr < cc).astype(jnp.float32)
    r2 = lax.broadcasted_iota(jnp.int32, (_HR, _HR), 0)
    c2 = lax.broadcasted_iota(jnp.int32, (_HR, _HR), 1)
    l_strict = (c2 < r2).astype(jnp.float32)
    m1 = lax.dot(l_strict, C, precision=lax.Precision.HIGHEST)
    rowpre = jnp.sum(m1, axis=1, keepdims=True)
    rowcum_excl = lax.dot(C, t_strict, precision=lax.Precision.HIGHEST)
    p_excl = rowpre + rowcum_excl
    total = jnp.sum(C)
    k = k_ref[0, 0]
    mask_le = (p_excl <= (total - k)).astype(jnp.float32)
    b_ref[0, 0] = jnp.sum(mask_le) - 1.0
    above_ref[0, 0] = jnp.sum(C * (1.0 - mask_le))


_scan_call = pl.pallas_call(
    _scan_body,
    in_specs=[
        pl.BlockSpec((_NC * _NREP, _HR, _HCOL), lambda: (0, 0, 0)),
        pl.BlockSpec((1, 1), lambda: (0, 0), memory_space=pltpu.SMEM),
    ],
    out_specs=[
        pl.BlockSpec((1, 1), lambda: (0, 0), memory_space=pltpu.SMEM),
        pl.BlockSpec((1, 1), lambda: (0, 0), memory_space=pltpu.SMEM),
    ],
    out_shape=[
        jax.ShapeDtypeStruct((1, 1), jnp.float32),
        jax.ShapeDtypeStruct((1, 1), jnp.float32),
    ],
)


def kernel(logits, labels):
    labels = labels.astype(jnp.int32)
    loss, cnt_gt, sum_gt = _loss_call(logits, labels)
    loss_flat = loss.reshape(_NPIX)

    p1 = jnp.full((16,), -1, dtype=jnp.int32)
    zeros_h = jnp.zeros((_NREP * _NBINS,), dtype=jnp.float32)
    hist = _get_hist_call()
    cnt1, _ = hist(loss_flat, p1, zeros_h)
    k1 = jnp.full((1, 1), float(_N_MIN), dtype=jnp.float32)
    b1f, above1 = _scan_call(cnt1.reshape(_NC * _NREP, _HR, _HCOL), k1)
    b1 = b1f[0, 0].astype(jnp.int32)
    k2 = _N_MIN - above1

    p2 = jnp.full((16,), 1, dtype=jnp.int32) * b1
    cnt2, psum1 = hist(loss_flat, p2, zeros_h)
    sumab1 = jnp.sum(psum1)
    b2f, above2 = _scan_call(cnt2.reshape(_NC * _NREP, _HR, _HCOL), k2)
    b2 = b2f[0, 0].astype(jnp.int32)

    p3 = jnp.concatenate([p2, jnp.full((16,), 1, dtype=jnp.int32) * b2])
    psum2 = _get_sum_call()(loss_flat, p3)
    sumab2 = jnp.sum(psum2)

    v_cut = lax.bitcast_convert_type(
        jnp.left_shift(b1, 16) | b2, jnp.float32)
    remaining = k2[0, 0] - above2[0, 0]
    sum_top = sumab1 + sumab2 + remaining * v_cut
    mean_b = sum_top / jnp.float32(_N_MIN)

    cg = cnt_gt[0, 0]
    mean_a = sum_gt[0, 0] / jnp.maximum(cg, 1.0)
    return jnp.where(cg > jnp.float32(_N_MIN), mean_a, mean_b)


# speedup vs baseline: 1.1555x; 1.1555x over previous
import functools

import jax
import jax.numpy as jnp
from jax import lax
from jax.experimental import pallas as pl
from jax.experimental.pallas import tpu as pltpu
from jax.experimental.pallas import tpu_sc as plsc

_THRESH_NLOG = 0.35667494393873245
_N_MIN = 65536

_N, _C, _H, _W = 4, 19, 512, 512
_NPIX = _N * _H * _W

_ROWS = 64
_RGRID = _H // _ROWS


def _loss_body(logits_ref, labels_ref, loss_ref, cnt_ref, sum_ref):
    x = logits_ref[0]
    m = jnp.max(x, axis=0)
    e = jnp.exp(x - m[None])
    s = jnp.sum(e, axis=0)
    lse = m + jnp.log(s)
    lbl = labels_ref[0]
    cls = lax.broadcasted_iota(jnp.int32, x.shape, 0)
    pick = jnp.sum(jnp.where(cls == lbl[None], x, 0.0), axis=0)
    loss = jnp.maximum(lse - pick, 0.0)
    loss_ref[0] = loss
    mask = loss > _THRESH_NLOG
    c = jnp.sum(mask.astype(jnp.float32))
    sv = jnp.sum(jnp.where(mask, loss, 0.0))
    first = jnp.logical_and(pl.program_id(0) == 0, pl.program_id(1) == 0)

    @pl.when(first)
    def _():
        cnt_ref[0, 0] = c
        sum_ref[0, 0] = sv

    @pl.when(jnp.logical_not(first))
    def _():
        cnt_ref[0, 0] = cnt_ref[0, 0] + c
        sum_ref[0, 0] = sum_ref[0, 0] + sv


_loss_call = pl.pallas_call(
    _loss_body,
    grid=(_N, _RGRID),
    in_specs=[
        pl.BlockSpec((1, _C, _ROWS, _W), lambda n, r: (n, 0, r, 0)),
        pl.BlockSpec((1, _ROWS, _W), lambda n, r: (n, r, 0)),
    ],
    out_specs=[
        pl.BlockSpec((1, _ROWS, _W), lambda n, r: (n, r, 0)),
        pl.BlockSpec((1, 1), lambda n, r: (0, 0), memory_space=pltpu.SMEM),
        pl.BlockSpec((1, 1), lambda n, r: (0, 0), memory_space=pltpu.SMEM),
    ],
    out_shape=[
        jax.ShapeDtypeStruct((_N, _H, _W), jnp.float32),
        jax.ShapeDtypeStruct((1, 1), jnp.float32),
        jax.ShapeDtypeStruct((1, 1), jnp.float32),
    ],
)

_NC, _NS = 2, 16
_NW = _NC * _NS
_PER_W = _NPIX // _NW
_CH = 4096
_NCHUNK = _PER_W // _CH
_NPAIR = _NCHUNK // 2
_NBINS = 65536
_NREP = 1
_BINS_PER_TILE = _NREP * _NBINS // _NS


def _sc_hist_body(loss_hbm, param_hbm, zeros_hbm, cnt_out, psum_out,
                  pbuf, dbuf_a, dbuf_b, ibuf_a, ibuf_b, cbuf_a, cbuf_b,
                  abuf, cnt_sh, sem_a, sem_b):
    c = lax.axis_index("c")
    s = lax.axis_index("s")
    wid = s * _NC + c
    sl = pl.ds(s * _BINS_PER_TILE, _BINS_PER_TILE)
    pltpu.sync_copy(zeros_hbm.at[sl], cnt_sh.at[sl])
    pltpu.sync_copy(param_hbm, pbuf)
    plsc.subcore_barrier()

    p = pbuf[...]
    is_p1 = p < 0
    base = wid * _PER_W
    lanes = lax.iota(jnp.int32, 16)
    rep_off = (s % _NREP) * _NBINS
    abuf[...] = jnp.zeros((16,), jnp.float32)
    dbufs = (dbuf_a, dbuf_b)
    ibufs = (ibuf_a, ibuf_b)
    cbufs = (cbuf_a, cbuf_b)
    sems = (sem_a, sem_b)

    def process(off0, db, ib, cb):
        def vec_body(k, acc):
            for u in range(4):
                off = k * 64 + u * 16
                v = db[pl.ds(off, 16)]
                bits = lax.bitcast_convert_type(v, jnp.int32)
                hi = lax.shift_right_logical(bits, 16)
                lo = lax.bitwise_and(bits, jnp.int32(0xFFFF))
                match = jnp.logical_or(is_p1, hi == p)
                pos = lax.bitwise_and(off0 + off + lanes, jnp.int32(0xFFFF))
                idx = jnp.where(match, jnp.where(is_p1, hi, lo), pos) + rep_off
                ib[pl.ds(off, 16)] = idx
                cb[pl.ds(off, 16)] = jnp.where(match, 1.0, 0.0)
                acc = acc + jnp.where(hi > p, v, 0.0)
            return acc

        acc = lax.fori_loop(0, _CH // 64, vec_body, jnp.zeros((16,), jnp.float32))
        abuf[...] = abuf[...] + acc
        pltpu.sync_copy(cb, cnt_sh.at[ib], add=True)

    def start_load(ci, which):
        ci = jnp.minimum(ci, _NCHUNK - 1)
        pltpu.async_copy(loss_hbm.at[pl.ds(base + ci * _CH, _CH)],
                         dbufs[which], sems[which])

    def wait_load(which):
        pltpu.make_async_copy(loss_hbm.at[pl.ds(0, _CH)],
                              dbufs[which], sems[which]).wait()

    start_load(0, 0)

    def pair_body(i, carry):
        start_load(2 * i + 1, 1)
        wait_load(0)
        process(base + (2 * i) * _CH, dbuf_a, ibuf_a, cbuf_a)
        start_load(2 * i + 2, 0)
        wait_load(1)
        process(base + (2 * i + 1) * _CH, dbuf_b, ibuf_b, cbuf_b)
        return carry

    lax.fori_loop(0, _NPAIR, pair_body, 0)
    wait_load(0)
    pltpu.sync_copy(abuf, psum_out.at[c, s])
    plsc.subcore_barrier()

    @pl.when(s == 0)
    def _():
        pltpu.sync_copy(cnt_sh, cnt_out.at[c])


def _sc_sum_body(loss_hbm, param_hbm, psum_out,
                 pbuf, dbuf_a, dbuf_b, abuf, sem_a, sem_b):
    c = lax.axis_index("c")
    s = lax.axis_index("s")
    wid = s * _NC + c
    pltpu.sync_copy(param_hbm, pbuf)
    b1 = pbuf[pl.ds(0, 16)]
    b2 = pbuf[pl.ds(16, 16)]
    base = wid * _PER_W
    dbufs = (dbuf_a, dbuf_b)
    sems = (sem_a, sem_b)

    def process(db, acc0):
        def vec_body(k, acc):
            for u in range(4):
                off = k * 64 + u * 16
                v = db[pl.ds(off, 16)]
                bits = lax.bitcast_convert_type(v, jnp.int32)
                hi = lax.shift_right_logical(bits, 16)
                lo = lax.bitwise_and(bits, jnp.int32(0xFFFF))
                m = jnp.logical_and(hi == b1, lo > b2)
                acc = acc + jnp.where(m, v, 0.0)
            return acc

        return lax.fori_loop(0, _CH // 64, vec_body, acc0)

    def start_load(ci, which):
        ci = jnp.minimum(ci, _NCHUNK - 1)
        pltpu.async_copy(loss_hbm.at[pl.ds(base + ci * _CH, _CH)],
                         dbufs[which], sems[which])

    def wait_load(which):
        pltpu.make_async_copy(loss_hbm.at[pl.ds(0, _CH)],
                              dbufs[which], sems[which]).wait()

    start_load(0, 0)

    def pair_body(i, acc):
        start_load(2 * i + 1, 1)
        wait_load(0)
        acc = process(dbuf_a, acc)
        start_load(2 * i + 2, 0)
        wait_load(1)
        acc = process(dbuf_b, acc)
        return acc

    acc = lax.fori_loop(0, _NPAIR, pair_body, jnp.zeros((16,), jnp.float32))
    wait_load(0)
    abuf[...] = acc
    pltpu.sync_copy(abuf, psum_out.at[c, s])


@functools.lru_cache(maxsize=1)
def _get_hist_call():
    return pl.kernel(
        _sc_hist_body,
        mesh=plsc.VectorSubcoreMesh(core_axis_name="c", subcore_axis_name="s",
                                    num_cores=_NC, num_subcores=_NS),
        out_type=[
            jax.ShapeDtypeStruct((_NC, _NREP * _NBINS), jnp.float32),
            jax.ShapeDtypeStruct((_NC, _NS, 16), jnp.float32),
        ],
        scratch_types=[
            pltpu.VMEM((16,), jnp.int32),
            pltpu.VMEM((_CH,), jnp.float32),
            pltpu.VMEM((_CH,), jnp.float32),
            pltpu.VMEM((_CH,), jnp.int32),
            pltpu.VMEM((_CH,), jnp.int32),
            pltpu.VMEM((_CH,), jnp.float32),
            pltpu.VMEM((_CH,), jnp.float32),
            pltpu.VMEM((16,), jnp.float32),
            pltpu.VMEM_SHARED((_NREP * _NBINS,), jnp.float32),
            pltpu.SemaphoreType.DMA,
            pltpu.SemaphoreType.DMA,
        ],
    )


@functools.lru_cache(maxsize=1)
def _get_sum_call():
    return pl.kernel(
        _sc_sum_body,
        mesh=plsc.VectorSubcoreMesh(core_axis_name="c", subcore_axis_name="s",
                                    num_cores=_NC, num_subcores=_NS),
        out_type=jax.ShapeDtypeStruct((_NC, _NS, 16), jnp.float32),
        scratch_types=[
            pltpu.VMEM((32,), jnp.int32),
            pltpu.VMEM((_CH,), jnp.float32),
            pltpu.VMEM((_CH,), jnp.float32),
            pltpu.VMEM((16,), jnp.float32),
            pltpu.SemaphoreType.DMA,
            pltpu.SemaphoreType.DMA,
        ],
    )

_HR, _HCOL = 512, 128


def _scan_body(cnt_ref, k_ref, b_ref, above_ref):
    C = jnp.sum(cnt_ref[...], axis=0)
    rr = lax.broadcasted_iota(jnp.int32, (_HCOL, _HCOL), 0)
    cc = lax.broadcasted_iota(jnp.int32, (_HCOL, _HCOL), 1)
    t_strict = (rr < cc).astype(jnp.float32)
    r2 = lax.broadcasted_iota(jnp.int32, (_HR, _HR), 0)
    c2 = lax.broadcasted_iota(jnp.int32, (_HR, _HR), 1)
    l_strict = (c2 < r2).astype(jnp.float32)
    m1 = lax.dot(l_strict, C, precision=lax.Precision.HIGHEST)
    rowpre = jnp.sum(m1, axis=1, keepdims=True)
    rowcum_excl = lax.dot(C, t_strict, precision=lax.Precision.HIGHEST)
    p_excl = rowpre + rowcum_excl
    total = jnp.sum(C)
    k = k_ref[0, 0]
    mask_le = (p_excl <= (total - k)).astype(jnp.float32)
    b_ref[0, 0] = jnp.sum(mask_le) - 1.0
    above_ref[0, 0] = jnp.sum(C * (1.0 - mask_le))


_scan_call = pl.pallas_call(
    _scan_body,
    in_specs=[
        pl.BlockSpec((_NC * _NREP, _HR, _HCOL), lambda: (0, 0, 0)),
        pl.BlockSpec((1, 1), lambda: (0, 0), memory_space=pltpu.SMEM),
    ],
    out_specs=[
        pl.BlockSpec((1, 1), lambda: (0, 0), memory_space=pltpu.SMEM),
        pl.BlockSpec((1, 1), lambda: (0, 0), memory_space=pltpu.SMEM),
    ],
    out_shape=[
        jax.ShapeDtypeStruct((1, 1), jnp.float32),
        jax.ShapeDtypeStruct((1, 1), jnp.float32),
    ],
)


def kernel(logits, labels):
    labels = labels.astype(jnp.int32)
    loss, cnt_gt, sum_gt = _loss_call(logits, labels)
    loss_flat = loss.reshape(_NPIX)

    p1 = jnp.full((16,), -1, dtype=jnp.int32)
    zeros_h = jnp.zeros((_NREP * _NBINS,), dtype=jnp.float32)
    hist = _get_hist_call()
    cnt1, _ = hist(loss_flat, p1, zeros_h)
    k1 = jnp.full((1, 1), float(_N_MIN), dtype=jnp.float32)
    b1f, above1 = _scan_call(cnt1.reshape(_NC * _NREP, _HR, _HCOL), k1)
    b1 = b1f[0, 0].astype(jnp.int32)
    k2 = _N_MIN - above1

    p2 = jnp.full((16,), 1, dtype=jnp.int32) * b1
    cnt2, psum1 = hist(loss_flat, p2, zeros_h)
    sumab1 = jnp.sum(psum1)
    b2f, above2 = _scan_call(cnt2.reshape(_NC * _NREP, _HR, _HCOL), k2)
    b2 = b2f[0, 0].astype(jnp.int32)

    p3 = jnp.concatenate([p2, jnp.full((16,), 1, dtype=jnp.int32) * b2])
    psum2 = _get_sum_call()(loss_flat, p3)
    sumab2 = jnp.sum(psum2)

    v_cut = lax.bitcast_convert_type(
        jnp.left_shift(b1, 16) | b2, jnp.float32)
    remaining = k2[0, 0] - above2[0, 0]
    sum_top = sumab1 + sumab2 + remaining * v_cut
    mean_b = sum_top / jnp.float32(_N_MIN)

    cg = cnt_gt[0, 0]
    mean_a = sum_gt[0, 0] / jnp.maximum(cg, 1.0)
    return jnp.where(cg > jnp.float32(_N_MIN), mean_a, mean_b)


# speedup vs baseline: 1.2116x; 1.0485x over previous
import functools

import jax
import jax.numpy as jnp
from jax import lax
from jax.experimental import pallas as pl
from jax.experimental.pallas import tpu as pltpu
from jax.experimental.pallas import tpu_sc as plsc

_THRESH_NLOG = 0.35667494393873245
_N_MIN = 65536

_N, _C, _H, _W = 4, 19, 512, 512
_NPIX = _N * _H * _W

_ROWS = 128
_RGRID = _H // _ROWS


def _loss_body(logits_ref, labels_ref, loss_ref, cnt_ref, sum_ref):
    x = logits_ref[0]
    m = jnp.max(x, axis=0)
    e = jnp.exp(x - m[None])
    s = jnp.sum(e, axis=0)
    lse = m + jnp.log(s)
    lbl = labels_ref[0]
    cls = lax.broadcasted_iota(jnp.int32, x.shape, 0)
    pick = jnp.sum(jnp.where(cls == lbl[None], x, 0.0), axis=0)
    loss = jnp.maximum(lse - pick, 0.0)
    loss_ref[0] = loss
    mask = loss > _THRESH_NLOG
    c = jnp.sum(mask.astype(jnp.float32))
    sv = jnp.sum(jnp.where(mask, loss, 0.0))
    first = jnp.logical_and(pl.program_id(0) == 0, pl.program_id(1) == 0)

    @pl.when(first)
    def _():
        cnt_ref[0, 0] = c
        sum_ref[0, 0] = sv

    @pl.when(jnp.logical_not(first))
    def _():
        cnt_ref[0, 0] = cnt_ref[0, 0] + c
        sum_ref[0, 0] = sum_ref[0, 0] + sv


_loss_call = pl.pallas_call(
    _loss_body,
    grid=(_N, _RGRID),
    in_specs=[
        pl.BlockSpec((1, _C, _ROWS, _W), lambda n, r: (n, 0, r, 0)),
        pl.BlockSpec((1, _ROWS, _W), lambda n, r: (n, r, 0)),
    ],
    out_specs=[
        pl.BlockSpec((1, _ROWS, _W), lambda n, r: (n, r, 0)),
        pl.BlockSpec((1, 1), lambda n, r: (0, 0), memory_space=pltpu.SMEM),
        pl.BlockSpec((1, 1), lambda n, r: (0, 0), memory_space=pltpu.SMEM),
    ],
    out_shape=[
        jax.ShapeDtypeStruct((_N, _H, _W), jnp.float32),
        jax.ShapeDtypeStruct((1, 1), jnp.float32),
        jax.ShapeDtypeStruct((1, 1), jnp.float32),
    ],
)

_NC, _NS = 2, 16
_NW = _NC * _NS
_PER_W = _NPIX // _NW
_CH = 4096
_NCHUNK = _PER_W // _CH
_NPAIR = _NCHUNK // 2
_NBINS = 65536
_NREP = 1
_BINS_PER_TILE = _NREP * _NBINS // _NS


def _sc_hist_body(loss_hbm, param_hbm, zeros_hbm, cnt_out, psum_out,
                  pbuf, dbuf_a, dbuf_b, ibuf_a, ibuf_b, cbuf_a, cbuf_b,
                  abuf, cnt_sh, sem_a, sem_b):
    c = lax.axis_index("c")
    s = lax.axis_index("s")
    wid = s * _NC + c
    sl = pl.ds(s * _BINS_PER_TILE, _BINS_PER_TILE)
    pltpu.sync_copy(zeros_hbm.at[sl], cnt_sh.at[sl])
    pltpu.sync_copy(param_hbm, pbuf)
    plsc.subcore_barrier()

    p = pbuf[...]
    is_p1 = p < 0
    base = wid * _PER_W
    lanes = lax.iota(jnp.int32, 16)
    rep_off = (s % _NREP) * _NBINS
    abuf[...] = jnp.zeros((16,), jnp.float32)
    dbufs = (dbuf_a, dbuf_b)
    ibufs = (ibuf_a, ibuf_b)
    cbufs = (cbuf_a, cbuf_b)
    sems = (sem_a, sem_b)

    def process(off0, db, ib, cb):
        def vec_body(k, acc):
            for u in range(4):
                off = k * 64 + u * 16
                v = db[pl.ds(off, 16)]
                bits = lax.bitcast_convert_type(v, jnp.int32)
                hi = lax.shift_right_logical(bits, 16)
                lo = lax.bitwise_and(bits, jnp.int32(0xFFFF))
                match = jnp.logical_or(is_p1, hi == p)
                pos = lax.bitwise_and(off0 + off + lanes, jnp.int32(0xFFFF))
                idx = jnp.where(match, jnp.where(is_p1, hi, lo), pos) + rep_off
                ib[pl.ds(off, 16)] = idx
                cb[pl.ds(off, 16)] = jnp.where(match, 1.0, 0.0)
                acc = acc + jnp.where(hi > p, v, 0.0)
            return acc

        acc = lax.fori_loop(0, _CH // 64, vec_body, jnp.zeros((16,), jnp.float32))
        abuf[...] = abuf[...] + acc
        pltpu.sync_copy(cb, cnt_sh.at[ib], add=True)

    def start_load(ci, which):
        ci = jnp.minimum(ci, _NCHUNK - 1)
        pltpu.async_copy(loss_hbm.at[pl.ds(base + ci * _CH, _CH)],
                         dbufs[which], sems[which])

    def wait_load(which):
        pltpu.make_async_copy(loss_hbm.at[pl.ds(0, _CH)],
                              dbufs[which], sems[which]).wait()

    start_load(0, 0)

    def pair_body(i, carry):
        start_load(2 * i + 1, 1)
        wait_load(0)
        process(base + (2 * i) * _CH, dbuf_a, ibuf_a, cbuf_a)
        start_load(2 * i + 2, 0)
        wait_load(1)
        process(base + (2 * i + 1) * _CH, dbuf_b, ibuf_b, cbuf_b)
        return carry

    lax.fori_loop(0, _NPAIR, pair_body, 0)
    wait_load(0)
    pltpu.sync_copy(abuf, psum_out.at[c, s])
    plsc.subcore_barrier()

    @pl.when(s == 0)
    def _():
        pltpu.sync_copy(cnt_sh, cnt_out.at[c])


def _sc_sum_body(loss_hbm, param_hbm, psum_out,
                 pbuf, dbuf_a, dbuf_b, abuf, sem_a, sem_b):
    c = lax.axis_index("c")
    s = lax.axis_index("s")
    wid = s * _NC + c
    pltpu.sync_copy(param_hbm, pbuf)
    b1 = pbuf[pl.ds(0, 16)]
    b2 = pbuf[pl.ds(16, 16)]
    base = wid * _PER_W
    dbufs = (dbuf_a, dbuf_b)
    sems = (sem_a, sem_b)

    def process(db, acc0):
        def vec_body(k, acc):
            for u in range(4):
                off = k * 64 + u * 16
                v = db[pl.ds(off, 16)]
                bits = lax.bitcast_convert_type(v, jnp.int32)
                hi = lax.shift_right_logical(bits, 16)
                lo = lax.bitwise_and(bits, jnp.int32(0xFFFF))
                m = jnp.logical_and(hi == b1, lo > b2)
                acc = acc + jnp.where(m, v, 0.0)
            return acc

        return lax.fori_loop(0, _CH // 64, vec_body, acc0)

    def start_load(ci, which):
        ci = jnp.minimum(ci, _NCHUNK - 1)
        pltpu.async_copy(loss_hbm.at[pl.ds(base + ci * _CH, _CH)],
                         dbufs[which], sems[which])

    def wait_load(which):
        pltpu.make_async_copy(loss_hbm.at[pl.ds(0, _CH)],
                              dbufs[which], sems[which]).wait()

    start_load(0, 0)

    def pair_body(i, acc):
        start_load(2 * i + 1, 1)
        wait_load(0)
        acc = process(dbuf_a, acc)
        start_load(2 * i + 2, 0)
        wait_load(1)
        acc = process(dbuf_b, acc)
        return acc

    acc = lax.fori_loop(0, _NPAIR, pair_body, jnp.zeros((16,), jnp.float32))
    wait_load(0)
    abuf[...] = acc
    pltpu.sync_copy(abuf, psum_out.at[c, s])


@functools.lru_cache(maxsize=1)
def _get_hist_call():
    return pl.kernel(
        _sc_hist_body,
        mesh=plsc.VectorSubcoreMesh(core_axis_name="c", subcore_axis_name="s",
                                    num_cores=_NC, num_subcores=_NS),
        out_type=[
            jax.ShapeDtypeStruct((_NC, _NREP * _NBINS), jnp.float32),
            jax.ShapeDtypeStruct((_NC, _NS, 16), jnp.float32),
        ],
        scratch_types=[
            pltpu.VMEM((16,), jnp.int32),
            pltpu.VMEM((_CH,), jnp.float32),
            pltpu.VMEM((_CH,), jnp.float32),
            pltpu.VMEM((_CH,), jnp.int32),
            pltpu.VMEM((_CH,), jnp.int32),
            pltpu.VMEM((_CH,), jnp.float32),
            pltpu.VMEM((_CH,), jnp.float32),
            pltpu.VMEM((16,), jnp.float32),
            pltpu.VMEM_SHARED((_NREP * _NBINS,), jnp.float32),
            pltpu.SemaphoreType.DMA,
            pltpu.SemaphoreType.DMA,
        ],
    )


@functools.lru_cache(maxsize=1)
def _get_sum_call():
    return pl.kernel(
        _sc_sum_body,
        mesh=plsc.VectorSubcoreMesh(core_axis_name="c", subcore_axis_name="s",
                                    num_cores=_NC, num_subcores=_NS),
        out_type=jax.ShapeDtypeStruct((_NC, _NS, 16), jnp.float32),
        scratch_types=[
            pltpu.VMEM((32,), jnp.int32),
            pltpu.VMEM((_CH,), jnp.float32),
            pltpu.VMEM((_CH,), jnp.float32),
            pltpu.VMEM((16,), jnp.float32),
            pltpu.SemaphoreType.DMA,
            pltpu.SemaphoreType.DMA,
        ],
    )

_HR, _HCOL = 512, 128


def _scan_body(cnt_ref, k_ref, b_ref, above_ref):
    C = jnp.sum(cnt_ref[...], axis=0)
    rr = lax.broadcasted_iota(jnp.int32, (_HCOL, _HCOL), 0)
    cc = lax.broadcasted_iota(jnp.int32, (_HCOL, _HCOL), 1)
    t_strict = (rr < cc).astype(jnp.float32)
    r2 = lax.broadcasted_iota(jnp.int32, (_HR, _HR), 0)
    c2 = lax.broadcasted_iota(jnp.int32, (_HR, _HR), 1)
    l_strict = (c2 < r2).astype(jnp.float32)
    m1 = lax.dot(l_strict, C, precision=lax.Precision.HIGHEST)
    rowpre = jnp.sum(m1, axis=1, keepdims=True)
    rowcum_excl = lax.dot(C, t_strict, precision=lax.Precision.HIGHEST)
    p_excl = rowpre + rowcum_excl
    total = jnp.sum(C)
    k = k_ref[0, 0]
    mask_le = (p_excl <= (total - k)).astype(jnp.float32)
    b_ref[0, 0] = jnp.sum(mask_le) - 1.0
    above_ref[0, 0] = jnp.sum(C * (1.0 - mask_le))


_scan_call = pl.pallas_call(
    _scan_body,
    in_specs=[
        pl.BlockSpec((_NC * _NREP, _HR, _HCOL), lambda: (0, 0, 0)),
        pl.BlockSpec((1, 1), lambda: (0, 0), memory_space=pltpu.SMEM),
    ],
    out_specs=[
        pl.BlockSpec((1, 1), lambda: (0, 0), memory_space=pltpu.SMEM),
        pl.BlockSpec((1, 1), lambda: (0, 0), memory_space=pltpu.SMEM),
    ],
    out_shape=[
        jax.ShapeDtypeStruct((1, 1), jnp.float32),
        jax.ShapeDtypeStruct((1, 1), jnp.float32),
    ],
)


def kernel(logits, labels):
    labels = labels.astype(jnp.int32)
    loss, cnt_gt, sum_gt = _loss_call(logits, labels)
    loss_flat = loss.reshape(_NPIX)

    p1 = jnp.full((16,), -1, dtype=jnp.int32)
    zeros_h = jnp.zeros((_NREP * _NBINS,), dtype=jnp.float32)
    hist = _get_hist_call()
    cnt1, _ = hist(loss_flat, p1, zeros_h)
    k1 = jnp.full((1, 1), float(_N_MIN), dtype=jnp.float32)
    b1f, above1 = _scan_call(cnt1.reshape(_NC * _NREP, _HR, _HCOL), k1)
    b1 = b1f[0, 0].astype(jnp.int32)
    k2 = _N_MIN - above1

    p2 = jnp.full((16,), 1, dtype=jnp.int32) * b1
    cnt2, psum1 = hist(loss_flat, p2, zeros_h)
    sumab1 = jnp.sum(psum1)
    b2f, above2 = _scan_call(cnt2.reshape(_NC * _NREP, _HR, _HCOL), k2)
    b2 = b2f[0, 0].astype(jnp.int32)

    p3 = jnp.concatenate([p2, jnp.full((16,), 1, dtype=jnp.int32) * b2])
    psum2 = _get_sum_call()(loss_flat, p3)
    sumab2 = jnp.sum(psum2)

    v_cut = lax.bitcast_convert_type(
        jnp.left_shift(b1, 16) | b2, jnp.float32)
    remaining = k2[0, 0] - above2[0, 0]
    sum_top = sumab1 + sumab2 + remaining * v_cut
    mean_b = sum_top / jnp.float32(_N_MIN)

    cg = cnt_gt[0, 0]
    mean_a = sum_gt[0, 0] / jnp.maximum(cg, 1.0)
    return jnp.where(cg > jnp.float32(_N_MIN), mean_a, mean_b)
